# baseline (speedup 1.0000x reference)
import functools
import jax
import jax.numpy as jnp
from jax import lax
from jax.experimental import pallas as pl
from jax.experimental.pallas import tpu as pltpu
from jax.experimental.pallas import tpu_sc as plsc

_VOCAB = 1000000
_D = 64
_LANES = 128
_SCALE = 8.0

_NC = 2
_NS = 16
_NW = _NC * _NS

_IDXW = 128
_HCHUNK = 128
_NPHASE = 2
_VPR = _D // 16
_TPC = _HCHUNK // 16


def _tc_scale(t_ref, o_ref):
    o_ref[:, :_D] = t_ref[...] * _SCALE


def _scaled_table(table):
    rows_blk = 8000
    grid = _VOCAB // rows_blk
    return pl.pallas_call(
        _tc_scale,
        grid=(grid,),
        in_specs=[pl.BlockSpec((rows_blk, _D), lambda i: (i, 0))],
        out_specs=pl.BlockSpec((rows_blk, _LANES), lambda i: (i, 0)),
        out_shape=jax.ShapeDtypeStruct((_VOCAB, _LANES), jnp.float32),
    )(table)


def _make_sc_gather(ntok):
    per_w = ntok // _NW
    per_phase = per_w // _NPHASE
    idx_rows = per_phase // _IDXW
    n_steps = per_phase // _HCHUNK
    mesh = plsc.VectorSubcoreMesh(core_axis_name="c", subcore_axis_name="s")

    @functools.partial(
        pl.kernel,
        mesh=mesh,
        out_type=jax.ShapeDtypeStruct((ntok // 16, 8, _LANES), jnp.float32),
        scratch_types=[
            pltpu.VMEM((idx_rows, _IDXW), jnp.int32),
            pltpu.VMEM((2, _HCHUNK, _LANES), jnp.float32),
            pltpu.VMEM((2, _TPC, 8, _LANES), jnp.float32),
            pltpu.SemaphoreType.DMA,
            pltpu.SemaphoreType.DMA,
            pltpu.SemaphoreType.DMA,
            pltpu.SemaphoreType.DMA,
        ],
    )
    def gather_kernel(table_hbm, idx_hbm, out_hbm, idx_v, rows_v, rows_c, g0, g1, w0, w1):
        wid = lax.axis_index("s") * _NC + lax.axis_index("c")
        base = wid * per_w
        gsem = (g0, g1)
        wsem = (w0, w1)

        def start_gather(h, b):
            pltpu.async_copy(table_hbm.at[idx_v.at[h]], rows_v.at[b], gsem[b])

        def wait_gather(b):
            pltpu.make_async_copy(
                table_hbm.at[idx_v.at[0]], rows_v.at[b], gsem[b]
            ).wait()

        def start_write(pbase, h, b):
            off8 = pl.multiple_of((pbase + h * _HCHUNK) // 16, _TPC)
            pltpu.async_copy(rows_c.at[b], out_hbm.at[pl.ds(off8, _TPC)], wsem[b])

        def wait_write(b):
            pltpu.make_async_copy(
                rows_c.at[b], out_hbm.at[pl.ds(0, _TPC)], wsem[b]
            ).wait()

        def pack_pairs(b):
            def cbody(t, carry):
                for u in range(8):
                    k = t * 8 + u
                    for j in range(_VPR):
                        sl = pl.ds(j * 16, 16)
                        sh = pl.ds(_D + j * 16, 16)
                        rows_c[b, t, u, sl] = rows_v[b, 2 * k, sl]
                        rows_c[b, t, u, sh] = rows_v[b, 2 * k + 1, sl]
                return carry

            lax.fori_loop(0, _TPC, cbody, 0)

        for q in range(_NPHASE):
            pbase = pl.multiple_of(base + q * per_phase, per_phase)
            row_off = pl.multiple_of(pbase // _IDXW, idx_rows)
            pltpu.sync_copy(idx_hbm.at[pl.ds(row_off, idx_rows)], idx_v)
            start_gather(0, 0)

            def step(h, carry):
                for u in range(2):
                    hh = h * 2 + u
                    b = u
                    nb = 1 - u
                    wait_gather(b)

                    @pl.when(hh >= 1)
                    def _():
                        wait_write(nb)

                    @pl.when(hh < n_steps - 1)
                    def _():
                        start_gather(hh + 1, nb)

                    pack_pairs(b)
                    start_write(pbase, hh, b)
                return carry

            lax.fori_loop(0, n_steps // 2, step, 0)
            wait_write((n_steps - 1) % 2)

    return gather_kernel


def kernel(x, table):
    b, s = x.shape
    ntok = b * s
    table8 = _scaled_table(table)
    idx2 = x.reshape(ntok // _IDXW, _IDXW)
    out4 = _make_sc_gather(ntok)(table8, idx2)
    return out4.reshape(b, s, _D)


# speedup vs baseline: 1.6885x; 1.6885x over previous
import functools
import jax
import jax.numpy as jnp
from jax import lax
from jax.experimental import pallas as pl
from jax.experimental.pallas import tpu as pltpu
from jax.experimental.pallas import tpu_sc as plsc

_VOCAB = 1000000
_D = 64
_LANES = 128
_SCALE = 8.0

_NC = 2
_NS = 16
_NW = _NC * _NS

_IDXW = 128
_HCHUNK = 256
_KPH = _HCHUNK // _IDXW
_NPHASE = 2


def _tc_scale(t_ref, o_ref):
    o_ref[:, :_D] = t_ref[...] * _SCALE


def _scaled_table(table):
    rows_blk = 25000
    grid = _VOCAB // rows_blk
    return pl.pallas_call(
        _tc_scale,
        grid=(grid,),
        in_specs=[pl.BlockSpec((rows_blk, _D), lambda i: (i, 0))],
        out_specs=pl.BlockSpec((rows_blk, _LANES), lambda i: (i, 0)),
        out_shape=jax.ShapeDtypeStruct((_VOCAB, _LANES), jnp.float32),
    )(table)


def _make_sc_gather(ntok):
    per_w = ntok // _NW
    per_phase = per_w // _NPHASE
    idx_rows = per_phase // _IDXW
    n_steps = per_phase // _HCHUNK
    mesh = plsc.VectorSubcoreMesh(core_axis_name="c", subcore_axis_name="s")

    @functools.partial(
        pl.kernel,
        mesh=mesh,
        out_type=jax.ShapeDtypeStruct((ntok, _LANES), jnp.float32),
        scratch_types=[
            pltpu.VMEM((idx_rows, _IDXW), jnp.int32),
            pltpu.VMEM((2, _HCHUNK, _LANES), jnp.float32),
            pltpu.SemaphoreType.DMA,
            pltpu.SemaphoreType.DMA,
            pltpu.SemaphoreType.DMA,
            pltpu.SemaphoreType.DMA,
        ],
    )
    def gather_kernel(table_hbm, idx_hbm, out_hbm, idx_v, rows_v, g0, g1, w0, w1):
        wid = lax.axis_index("s") * _NC + lax.axis_index("c")
        base = wid * per_w
        gsem = (g0, g1)
        wsem = (w0, w1)

        def start_gather(h, b):
            for j in range(_KPH):
                pltpu.async_copy(
                    table_hbm.at[idx_v.at[h * _KPH + j]],
                    rows_v.at[b, pl.ds(j * _IDXW, _IDXW)],
                    gsem[b],
                )

        def wait_gather(b):
            for j in range(_KPH):
                pltpu.make_async_copy(
                    table_hbm.at[idx_v.at[0]],
                    rows_v.at[b, pl.ds(j * _IDXW, _IDXW)],
                    gsem[b],
                ).wait()

        def start_write(pbase, h, b):
            off = pbase + h * _HCHUNK
            pltpu.async_copy(rows_v.at[b], out_hbm.at[pl.ds(off, _HCHUNK)], wsem[b])

        def wait_write(b):
            pltpu.make_async_copy(
                rows_v.at[b], out_hbm.at[pl.ds(0, _HCHUNK)], wsem[b]
            ).wait()

        for q in range(_NPHASE):
            pbase = pl.multiple_of(base + q * per_phase, per_phase)
            row_off = pl.multiple_of(pbase // _IDXW, idx_rows)
            pltpu.sync_copy(idx_hbm.at[pl.ds(row_off, idx_rows)], idx_v)
            start_gather(0, 0)

            def step(h, carry):
                for u in range(2):
                    hh = h * 2 + u
                    b = u
                    nb = 1 - u
                    wait_gather(b)

                    @pl.when(hh >= 1)
                    def _():
                        wait_write(nb)

                    @pl.when(hh < n_steps - 1)
                    def _():
                        start_gather(hh + 1, nb)

                    start_write(pbase, hh, b)
                return carry

            lax.fori_loop(0, n_steps // 2, step, 0)
            wait_write((n_steps - 1) % 2)

    return gather_kernel


def kernel(x, table):
    b, s = x.shape
    ntok = b * s
    table8 = _scaled_table(table)
    idx2 = x.reshape(ntok // _IDXW, _IDXW)
    out2 = _make_sc_gather(ntok)(table8, idx2)
    return out2[:, :_D].reshape(b, s, _D)
